# Initial kernel scaffold; baseline (speedup 1.0000x reference)
import functools

import jax
import jax.numpy as jnp
from jax import lax
from jax.experimental import pallas as pl
from jax.experimental.pallas import tpu as pltpu
from jax.experimental.pallas import tpu_sc as plsc

N_NODES = 10000
N_EDGES = 160000
FB = 128
NC, NS = 2, 16
EPT = N_EDGES // (NC * NS)
CH = 125
NCHUNK = EPT // CH
RPT = N_NODES // NS
BN_EPS = 1e-5


def _sc_agg(h, src2d, dst2d):
    @functools.partial(
        pl.kernel,
        mesh=plsc.VectorSubcoreMesh(core_axis_name="c", subcore_axis_name="s"),
        out_type=jax.ShapeDtypeStruct((NC, N_NODES, FB), jnp.float32),
        scratch_types=[
            pltpu.VMEM((NCHUNK, CH), jnp.int32),
            pltpu.VMEM((NCHUNK, CH), jnp.int32),
            pltpu.VMEM((CH, FB), jnp.float32),
            pltpu.VMEM_SHARED((N_NODES, FB), jnp.float32),
            pltpu.SemaphoreType.DMA,
        ],
    )
    def agg_kernel(h_hbm, src_hbm, dst_hbm, out_hbm, src_v, dst_v, rows_v,
                   acc_s, sem):
        c = lax.axis_index("c")
        s = lax.axis_index("s")
        tile = c * NS + s

        def zero_row(i, carry):
            for jj in range(FB // 16):
                rows_v[i, pl.ds(jj * 16, 16)] = jnp.zeros((16,), jnp.float32)
            return carry

        lax.fori_loop(0, CH, zero_row, 0)
        base = s * RPT
        for kk in range(RPT // CH):
            pltpu.sync_copy(rows_v, acc_s.at[pl.ds(base + kk * CH, CH)])

        pltpu.sync_copy(src_hbm.at[pl.ds(tile * NCHUNK, NCHUNK)], src_v)
        pltpu.sync_copy(dst_hbm.at[pl.ds(tile * NCHUNK, NCHUNK)], dst_v)
        plsc.subcore_barrier()

        def body(j, carry):
            pltpu.async_copy(h_hbm.at[src_v.at[j]], rows_v, sem).wait()
            pltpu.sync_copy(rows_v, acc_s.at[dst_v.at[j]], add=True)
            return carry

        lax.fori_loop(0, NCHUNK, body, 0)
        plsc.subcore_barrier()
        pltpu.sync_copy(acc_s.at[pl.ds(base, RPT)],
                        out_hbm.at[c, pl.ds(base, RPT)])

    return agg_kernel(h, src2d, dst2d)


def _mm(x, W):
    M, K = x.shape
    F = W.shape[1]
    BM = 1250

    def body(x_ref, w_ref, o_ref):
        o_ref[...] = jnp.dot(x_ref[...], w_ref[...],
                             preferred_element_type=jnp.float32)

    return pl.pallas_call(
        body,
        grid=(M // BM,),
        in_specs=[pl.BlockSpec((BM, K), lambda i: (i, 0)),
                  pl.BlockSpec((K, F), lambda i: (0, 0))],
        out_specs=pl.BlockSpec((BM, F), lambda i: (i, 0)),
        out_shape=jax.ShapeDtypeStruct((M, F), jnp.float32),
    )(x, W)


def _bn_relu(p_lo, p_hi, h, b, g, be):
    def body(plo_ref, phi_ref, h_ref, b_ref, g_ref, be_ref, o_ref):
        agg = jnp.concatenate([plo_ref[0] + plo_ref[1],
                               phi_ref[0] + phi_ref[1]], axis=1)
        z = agg + h_ref[...] + b_ref[...]
        mean = jnp.mean(z, axis=0, keepdims=True)
        zc = z - mean
        var = jnp.mean(zc * zc, axis=0, keepdims=True)
        zn = g_ref[...] * zc * lax.rsqrt(var + BN_EPS) + be_ref[...]
        o_ref[...] = jnp.maximum(zn, 0.0)

    N, F = h.shape
    return pl.pallas_call(
        body,
        out_shape=jax.ShapeDtypeStruct((N, F), jnp.float32),
    )(p_lo, p_hi, h, b.reshape(1, F), g.reshape(1, F), be.reshape(1, F))


def _final(p3, h3, b3):
    def body(p_ref, h_ref, b_ref, o_ref):
        o_ref[...] = p_ref[0] + p_ref[1] + h_ref[...] + b_ref[...]

    N, F = h3.shape
    return pl.pallas_call(
        body,
        out_shape=jax.ShapeDtypeStruct((N, F), jnp.float32),
    )(p3, h3, b3.reshape(1, F))


def kernel(x, edge_idx, W1, b1, g1, be1, W2, b2, g2, be2, W3, b3):
    ei = edge_idx.astype(jnp.int32)
    src = ei[0].reshape(NC * NS * NCHUNK, CH)
    dst = ei[1].reshape(NC * NS * NCHUNK, CH)

    h1 = _mm(x, W1)
    p1_lo = _sc_agg(h1[:, :FB], src, dst)
    p1_hi = _sc_agg(h1[:, FB:], src, dst)
    r1 = _bn_relu(p1_lo, p1_hi, h1, b1, g1, be1)

    h2 = _mm(r1, W2)
    p2_lo = _sc_agg(h2[:, :FB], src, dst)
    p2_hi = _sc_agg(h2[:, FB:], src, dst)
    r2 = _bn_relu(p2_lo, p2_hi, h2, b2, g2, be2)

    h3 = _mm(r2, W3)
    p3 = _sc_agg(h3, src, dst)
    return _final(p3, h3, b3)


# speedup vs baseline: 5.1939x; 5.1939x over previous
import functools

import jax
import jax.numpy as jnp
from jax import lax
from jax.experimental import pallas as pl
from jax.experimental.pallas import tpu as pltpu
from jax.experimental.pallas import tpu_sc as plsc

N_NODES = 10000
N_EDGES = 160000
FB = 128
NC, NS = 2, 16
EPT = N_EDGES // (NC * NS)
CH = 125
NCHUNK = EPT // CH
NPAD = 10240
RPT = NPAD // NS
ZR = 128
BN_EPS = 1e-5


def _sc_agg(h, src2d, dst2d):
    @functools.partial(
        pl.kernel,
        mesh=plsc.VectorSubcoreMesh(core_axis_name="c", subcore_axis_name="s"),
        out_type=jax.ShapeDtypeStruct((NC, NPAD, FB), jnp.float32),
        scratch_types=[
            pltpu.VMEM((NCHUNK, CH), jnp.int32),
            pltpu.VMEM((NCHUNK, CH), jnp.int32),
            pltpu.VMEM((CH, FB), jnp.float32),
            pltpu.VMEM((ZR, FB), jnp.float32),
            pltpu.VMEM_SHARED((NPAD, FB), jnp.float32),
            pltpu.SemaphoreType.DMA,
        ],
    )
    def agg_kernel(h_hbm, src_hbm, dst_hbm, out_hbm, src_v, dst_v, rows_v,
                   zrows_v, acc_s, sem):
        c = lax.axis_index("c")
        s = lax.axis_index("s")
        tile = c * NS + s

        def zero_row(i, carry):
            for jj in range(FB // 16):
                zrows_v[i, pl.ds(jj * 16, 16)] = jnp.zeros((16,), jnp.float32)
            return carry

        lax.fori_loop(0, ZR, zero_row, 0)
        base = s * RPT
        for kk in range(RPT // ZR):
            pltpu.sync_copy(zrows_v, acc_s.at[pl.ds(base + kk * ZR, ZR)])

        pltpu.sync_copy(src_hbm.at[pl.ds(tile * NCHUNK, NCHUNK)], src_v)
        pltpu.sync_copy(dst_hbm.at[pl.ds(tile * NCHUNK, NCHUNK)], dst_v)
        plsc.subcore_barrier()

        def body(j, carry):
            pltpu.async_copy(h_hbm.at[src_v.at[j]], rows_v, sem).wait()
            pltpu.sync_copy(rows_v, acc_s.at[dst_v.at[j]], add=True)
            return carry

        lax.fori_loop(0, NCHUNK, body, 0)
        plsc.subcore_barrier()
        pltpu.sync_copy(acc_s.at[pl.ds(base, RPT)],
                        out_hbm.at[c, pl.ds(base, RPT)])

    return agg_kernel(h, src2d, dst2d)[:, :N_NODES]


def _mm(x, W):
    M, K = x.shape
    F = W.shape[1]
    BM = 1000

    def body(x_ref, w_ref, o_ref):
        o_ref[...] = jnp.dot(x_ref[...], w_ref[...],
                             preferred_element_type=jnp.float32)

    return pl.pallas_call(
        body,
        grid=(M // BM,),
        in_specs=[pl.BlockSpec((BM, K), lambda i: (i, 0)),
                  pl.BlockSpec((K, F), lambda i: (0, 0))],
        out_specs=pl.BlockSpec((BM, F), lambda i: (i, 0)),
        out_shape=jax.ShapeDtypeStruct((M, F), jnp.float32),
    )(x, W)


def _bn_relu(p_lo, p_hi, h, b, g, be):
    def body(plo_ref, phi_ref, h_ref, b_ref, g_ref, be_ref, o_ref):
        agg = jnp.concatenate([plo_ref[0] + plo_ref[1],
                               phi_ref[0] + phi_ref[1]], axis=1)
        z = agg + h_ref[...] + b_ref[...]
        mean = jnp.mean(z, axis=0, keepdims=True)
        zc = z - mean
        var = jnp.mean(zc * zc, axis=0, keepdims=True)
        zn = g_ref[...] * zc * lax.rsqrt(var + BN_EPS) + be_ref[...]
        o_ref[...] = jnp.maximum(zn, 0.0)

    N, F = h.shape
    return pl.pallas_call(
        body,
        out_shape=jax.ShapeDtypeStruct((N, F), jnp.float32),
    )(p_lo, p_hi, h, b.reshape(1, F), g.reshape(1, F), be.reshape(1, F))


def _final(p3, h3, b3):
    def body(p_ref, h_ref, b_ref, o_ref):
        o_ref[...] = p_ref[0] + p_ref[1] + h_ref[...] + b_ref[...]

    N, F = h3.shape
    return pl.pallas_call(
        body,
        out_shape=jax.ShapeDtypeStruct((N, F), jnp.float32),
    )(p3, h3, b3.reshape(1, F))


def kernel(x, edge_idx, W1, b1, g1, be1, W2, b2, g2, be2, W3, b3):
    ei = edge_idx.astype(jnp.int32)
    src = ei[0].reshape(NC * NS * NCHUNK, CH)
    dst = ei[1].reshape(NC * NS * NCHUNK, CH)

    h1 = _mm(x, W1)
    p1_lo = _sc_agg(h1[:, :FB], src, dst)
    p1_hi = _sc_agg(h1[:, FB:], src, dst)
    r1 = _bn_relu(p1_lo, p1_hi, h1, b1, g1, be1)

    h2 = _mm(r1, W2)
    p2_lo = _sc_agg(h2[:, :FB], src, dst)
    p2_hi = _sc_agg(h2[:, FB:], src, dst)
    r2 = _bn_relu(p2_lo, p2_hi, h2, b2, g2, be2)

    h3 = _mm(r2, W3)
    p3 = _sc_agg(h3, src, dst)
    return _final(p3, h3, b3)


# speedup vs baseline: 6.3576x; 1.2241x over previous
import functools

import jax
import jax.numpy as jnp
from jax import lax
from jax.experimental import pallas as pl
from jax.experimental.pallas import tpu as pltpu
from jax.experimental.pallas import tpu_sc as plsc

N_NODES = 10000
N_EDGES = 160000
FB = 128
NC, NS = 2, 16
EPT = N_EDGES // (NC * NS)
CH = 125
NCHUNK = EPT // CH
NB = 2
NGROUP = NCHUNK // NB
NPAD = 10240
RPT = NPAD // NS
ZR = 32
BN_EPS = 1e-5


def _sc_agg(h_blocks, src2d, dst2d):
    nph = len(h_blocks)

    @functools.partial(
        pl.kernel,
        mesh=plsc.VectorSubcoreMesh(core_axis_name="c", subcore_axis_name="s"),
        out_type=jax.ShapeDtypeStruct((nph, NC, NPAD, FB), jnp.float32),
        scratch_types=[
            pltpu.VMEM((NCHUNK, CH), jnp.int32),
            pltpu.VMEM((NCHUNK, CH), jnp.int32),
            pltpu.VMEM((NB, CH, FB), jnp.float32),
            pltpu.VMEM((ZR, FB), jnp.float32),
            pltpu.VMEM_SHARED((NPAD, FB), jnp.float32),
        ] + [pltpu.SemaphoreType.DMA] * (2 * NB),
    )
    def agg_kernel(*refs):
        h_refs = refs[:nph]
        src_hbm, dst_hbm, out_hbm, src_v, dst_v, rows_v, zrows_v, acc_s = (
            refs[nph:nph + 8])
        gsem = refs[nph + 8:nph + 8 + NB]
        ssem = refs[nph + 8 + NB:nph + 8 + 2 * NB]

        c = lax.axis_index("c")
        s = lax.axis_index("s")
        tile = c * NS + s
        base = s * RPT

        pltpu.sync_copy(src_hbm.at[tile], src_v)
        pltpu.sync_copy(dst_hbm.at[tile], dst_v)

        def zero_row(i, carry):
            for jj in range(FB // 16):
                zrows_v[i, pl.ds(jj * 16, 16)] = jnp.zeros((16,), jnp.float32)
            return carry

        lax.fori_loop(0, ZR, zero_row, 0)

        for ph in range(nph):
            h_ref = h_refs[ph]
            for kk in range(RPT // ZR):
                pltpu.sync_copy(zrows_v, acc_s.at[pl.ds(base + kk * ZR, ZR)])
            plsc.subcore_barrier()

            pltpu.async_copy(h_ref.at[src_v.at[0]], rows_v.at[0], gsem[0])
            pltpu.async_copy(h_ref.at[src_v.at[1]], rows_v.at[1], gsem[1])

            def group(gi, carry):
                for b in range(NB):
                    j = gi * NB + b
                    ob = 1 - b
                    pltpu.make_async_copy(h_ref.at[src_v.at[j]],
                                          rows_v.at[b], gsem[b]).wait()

                    @pl.when(j >= 1)
                    def _drain_prev():
                        pltpu.make_async_copy(rows_v.at[ob],
                                              acc_s.at[dst_v.at[j - 1]],
                                              ssem[ob]).wait()

                    pltpu.async_copy(rows_v.at[b], acc_s.at[dst_v.at[j]],
                                     ssem[b], add=True)

                    @pl.when(jnp.logical_and(j >= 1, j + 1 < NCHUNK))
                    def _prefetch_next():
                        pltpu.async_copy(h_ref.at[src_v.at[j + 1]],
                                         rows_v.at[ob], gsem[ob])

                return carry

            lax.fori_loop(0, NGROUP, group, 0)

            jl = NCHUNK - 1
            pltpu.make_async_copy(rows_v.at[jl % 2], acc_s.at[dst_v.at[jl]],
                                  ssem[jl % 2]).wait()
            plsc.subcore_barrier()
            pltpu.sync_copy(acc_s.at[pl.ds(base, RPT)],
                            out_hbm.at[ph, c, pl.ds(base, RPT)])

    return agg_kernel(*h_blocks, src2d, dst2d)


def _mm(x, W):
    M, K = x.shape
    F = W.shape[1]
    BM = 1000

    def body(x_ref, w_ref, o_ref):
        o_ref[...] = jnp.dot(x_ref[...], w_ref[...],
                             preferred_element_type=jnp.float32)

    return pl.pallas_call(
        body,
        grid=(M // BM,),
        in_specs=[pl.BlockSpec((BM, K), lambda i: (i, 0)),
                  pl.BlockSpec((K, F), lambda i: (0, 0))],
        out_specs=pl.BlockSpec((BM, F), lambda i: (i, 0)),
        out_shape=jax.ShapeDtypeStruct((M, F), jnp.float32),
    )(x, W)


def _bn_relu(p, h, b, g, be):
    def body(p_ref, h_ref, b_ref, g_ref, be_ref, o_ref):
        agg = jnp.concatenate(
            [p_ref[0, 0] + p_ref[0, 1], p_ref[1, 0] + p_ref[1, 1]], axis=1)
        z = agg + h_ref[...] + b_ref[...]
        mean = jnp.mean(z, axis=0, keepdims=True)
        zc = z - mean
        var = jnp.mean(zc * zc, axis=0, keepdims=True)
        zn = g_ref[...] * zc * lax.rsqrt(var + BN_EPS) + be_ref[...]
        o_ref[...] = jnp.maximum(zn, 0.0)

    N, F = h.shape
    return pl.pallas_call(
        body,
        out_shape=jax.ShapeDtypeStruct((N, F), jnp.float32),
    )(p[:, :, :N_NODES], h, b.reshape(1, F), g.reshape(1, F),
      be.reshape(1, F))


def _final(p3, h3, b3):
    def body(p_ref, h_ref, b_ref, o_ref):
        o_ref[...] = p_ref[0, 0] + p_ref[0, 1] + h_ref[...] + b_ref[...]

    N, F = h3.shape
    return pl.pallas_call(
        body,
        out_shape=jax.ShapeDtypeStruct((N, F), jnp.float32),
    )(p3[:, :, :N_NODES], h3, b3.reshape(1, F))


def kernel(x, edge_idx, W1, b1, g1, be1, W2, b2, g2, be2, W3, b3):
    ei = edge_idx.astype(jnp.int32)
    src = ei[0].reshape(NC * NS, NCHUNK, CH)
    dst = ei[1].reshape(NC * NS, NCHUNK, CH)

    h1 = _mm(x, W1)
    p1 = _sc_agg([h1[:, :FB], h1[:, FB:]], src, dst)
    r1 = _bn_relu(p1, h1, b1, g1, be1)

    h2 = _mm(r1, W2)
    p2 = _sc_agg([h2[:, :FB], h2[:, FB:]], src, dst)
    r2 = _bn_relu(p2, h2, b2, g2, be2)

    h3 = _mm(r2, W3)
    p3 = _sc_agg([h3], src, dst)
    return _final(p3, h3, b3)


# speedup vs baseline: 7.0341x; 1.1064x over previous
import functools

import jax
import jax.numpy as jnp
from jax import lax
from jax.experimental import pallas as pl
from jax.experimental.pallas import tpu as pltpu
from jax.experimental.pallas import tpu_sc as plsc

N_NODES = 10000
N_EDGES = 160000
FB = 128
NC, NS = 2, 16
EPT = N_EDGES // (NC * NS)
CH = 125
NCHUNK = EPT // CH
NB = 2
NGROUP = NCHUNK // NB
NPAD = 10240
RPT = NPAD // NS
ZR = 32
BN_EPS = 1e-5


def _sc_agg(h_blocks, src3d, dst3d):
    nph = len(h_blocks)

    @functools.partial(
        pl.kernel,
        mesh=plsc.VectorSubcoreMesh(core_axis_name="c", subcore_axis_name="s"),
        out_type=tuple(
            jax.ShapeDtypeStruct((NC, NPAD, FB), jnp.float32)
            for _ in range(nph)),
        scratch_types=[
            pltpu.VMEM((NCHUNK, CH), jnp.int32),
            pltpu.VMEM((NCHUNK, CH), jnp.int32),
            pltpu.VMEM((NB, CH, FB), jnp.float32),
            pltpu.VMEM((ZR, FB), jnp.float32),
            pltpu.VMEM_SHARED((NPAD, FB), jnp.float32),
        ] + [pltpu.SemaphoreType.DMA] * (2 * NB),
    )
    def agg_kernel(*refs):
        h_refs = refs[:nph]
        src_hbm, dst_hbm = refs[nph:nph + 2]
        out_refs = refs[nph + 2:nph + 2 + nph]
        src_v, dst_v, rows_v, zrows_v, acc_s = refs[nph + 2 + nph:
                                                    nph + 2 + nph + 5]
        gsem = refs[nph + 2 + nph + 5:nph + 2 + nph + 5 + NB]
        ssem = refs[nph + 2 + nph + 5 + NB:nph + 2 + nph + 5 + 2 * NB]

        c = lax.axis_index("c")
        s = lax.axis_index("s")
        tile = c * NS + s
        base = s * RPT

        pltpu.sync_copy(src_hbm.at[tile], src_v)
        pltpu.sync_copy(dst_hbm.at[tile], dst_v)

        def zero_row(i, carry):
            for jj in range(FB // 16):
                zrows_v[i, pl.ds(jj * 16, 16)] = jnp.zeros((16,), jnp.float32)
            return carry

        lax.fori_loop(0, ZR, zero_row, 0)

        for ph in range(nph):
            h_ref = h_refs[ph]
            for kk in range(RPT // ZR):
                pltpu.sync_copy(zrows_v, acc_s.at[pl.ds(base + kk * ZR, ZR)])
            plsc.subcore_barrier()

            pltpu.async_copy(h_ref.at[src_v.at[0]], rows_v.at[0], gsem[0])
            pltpu.async_copy(h_ref.at[src_v.at[1]], rows_v.at[1], gsem[1])

            def group(gi, carry):
                for b in range(NB):
                    j = gi * NB + b
                    ob = 1 - b
                    pltpu.make_async_copy(h_ref.at[src_v.at[j]],
                                          rows_v.at[b], gsem[b]).wait()

                    @pl.when(j >= 1)
                    def _drain_prev():
                        pltpu.make_async_copy(rows_v.at[ob],
                                              acc_s.at[dst_v.at[j - 1]],
                                              ssem[ob]).wait()

                    pltpu.async_copy(rows_v.at[b], acc_s.at[dst_v.at[j]],
                                     ssem[b], add=True)

                    @pl.when(jnp.logical_and(j >= 1, j + 1 < NCHUNK))
                    def _prefetch_next():
                        pltpu.async_copy(h_ref.at[src_v.at[j + 1]],
                                         rows_v.at[ob], gsem[ob])

                return carry

            lax.fori_loop(0, NGROUP, group, 0)

            jl = NCHUNK - 1
            pltpu.make_async_copy(rows_v.at[jl % 2], acc_s.at[dst_v.at[jl]],
                                  ssem[jl % 2]).wait()
            plsc.subcore_barrier()
            pltpu.sync_copy(acc_s.at[pl.ds(base, RPT)],
                            out_refs[ph].at[c, pl.ds(base, RPT)])

    return agg_kernel(*h_blocks, src3d, dst3d)


def _mm_split(x_parts, W, split_out):
    M = x_parts[0].shape[0]
    K = sum(p.shape[1] for p in x_parts)
    F = W.shape[1]
    BM = 1000

    def body(*refs):
        x_refs = refs[:len(x_parts)]
        w_ref = refs[len(x_parts)]
        o_refs = refs[len(x_parts) + 1:]
        k0 = 0
        acc = None
        for xr in x_refs:
            kc = xr.shape[1]
            part = jnp.dot(xr[...], w_ref[pl.ds(k0, kc), :],
                           preferred_element_type=jnp.float32)
            acc = part if acc is None else acc + part
            k0 += kc
        if len(o_refs) == 1:
            o_refs[0][...] = acc
        else:
            for i, o_ref in enumerate(o_refs):
                o_ref[...] = acc[:, i * FB:(i + 1) * FB]

    n_out = F // FB if split_out else 1
    fo = FB if split_out else F
    out = pl.pallas_call(
        body,
        grid=(M // BM,),
        in_specs=[pl.BlockSpec((BM, p.shape[1]), lambda i: (i, 0))
                  for p in x_parts]
                 + [pl.BlockSpec((K, F), lambda i: (0, 0))],
        out_specs=[pl.BlockSpec((BM, fo), lambda i: (i, 0))] * n_out,
        out_shape=[jax.ShapeDtypeStruct((M, fo), jnp.float32)] * n_out,
    )(*x_parts, W)
    return out


def _bn_relu(p_lo, p_hi, h_lo, h_hi, b, g, be):
    def half(p_ref, h_ref, b_ref, g_ref, be_ref, o_ref):
        agg = p_ref[0, :N_NODES, :] + p_ref[1, :N_NODES, :]
        z = agg + h_ref[...] + b_ref[...]
        mean = jnp.mean(z, axis=0, keepdims=True)
        zc = z - mean
        var = jnp.mean(zc * zc, axis=0, keepdims=True)
        zn = g_ref[...] * zc * lax.rsqrt(var + BN_EPS) + be_ref[...]
        o_ref[...] = jnp.maximum(zn, 0.0)

    def body(plo_ref, phi_ref, hlo_ref, hhi_ref, b_ref, g_ref, be_ref,
             olo_ref, ohi_ref):
        half(plo_ref, hlo_ref, b_ref.at[:, pl.ds(0, FB)],
             g_ref.at[:, pl.ds(0, FB)], be_ref.at[:, pl.ds(0, FB)], olo_ref)
        half(phi_ref, hhi_ref, b_ref.at[:, pl.ds(FB, FB)],
             g_ref.at[:, pl.ds(FB, FB)], be_ref.at[:, pl.ds(FB, FB)],
             ohi_ref)

    F2 = 2 * FB
    return pl.pallas_call(
        body,
        out_shape=[jax.ShapeDtypeStruct((N_NODES, FB), jnp.float32)] * 2,
    )(p_lo, p_hi, h_lo, h_hi, b.reshape(1, F2), g.reshape(1, F2),
      be.reshape(1, F2))


def _final(p3, h3, b3):
    def body(p_ref, h_ref, b_ref, o_ref):
        o_ref[...] = (p_ref[0, :N_NODES, :] + p_ref[1, :N_NODES, :]
                      + h_ref[...] + b_ref[...])

    N, F = h3.shape
    return pl.pallas_call(
        body,
        out_shape=jax.ShapeDtypeStruct((N, F), jnp.float32),
    )(p3, h3, b3.reshape(1, F))


def kernel(x, edge_idx, W1, b1, g1, be1, W2, b2, g2, be2, W3, b3):
    ei = edge_idx.astype(jnp.int32)
    src = ei[0].reshape(NC * NS, NCHUNK, CH)
    dst = ei[1].reshape(NC * NS, NCHUNK, CH)

    h1_lo, h1_hi = _mm_split([x], W1, split_out=True)
    p1_lo, p1_hi = _sc_agg([h1_lo, h1_hi], src, dst)
    r1_lo, r1_hi = _bn_relu(p1_lo, p1_hi, h1_lo, h1_hi, b1, g1, be1)

    h2_lo, h2_hi = _mm_split([r1_lo, r1_hi], W2, split_out=True)
    p2_lo, p2_hi = _sc_agg([h2_lo, h2_hi], src, dst)
    r2_lo, r2_hi = _bn_relu(p2_lo, p2_hi, h2_lo, h2_hi, b2, g2, be2)

    (h3,) = _mm_split([r2_lo, r2_hi], W3, split_out=False)
    (p3,) = _sc_agg([h3], src, dst)
    return _final(p3, h3, b3)
